# baseline (speedup 1.0000x reference)
import functools

import jax
import jax.numpy as jnp
from jax import lax
from jax.experimental import pallas as pl
from jax.experimental.pallas import tpu as pltpu
from jax.experimental.pallas import tpu_sc as plsc

N = 10000
E = 320000
H = 128

NC = 2
NS = 16
NW = NC * NS
CH = 64
NCHUNK = 160
E_PAD = NW * NCHUNK * CH
DEPTH = 2
SEG = 16
NSEG = NCHUNK // SEG

N_ACC = 10112
ZROWS = N_ACC // NS


def _sc_aggregate(xp, srcq, dsti, zeros):
    mesh = plsc.VectorSubcoreMesh(core_axis_name="c", subcore_axis_name="s")

    @functools.partial(
        pl.kernel,
        out_type=jax.ShapeDtypeStruct((NC, N_ACC, H), jnp.float32),
        mesh=mesh,
        scratch_types=[
            pltpu.VMEM((SEG, CH), jnp.int32),
            pltpu.VMEM((SEG, 2 * CH), jnp.int32),
            [pltpu.VMEM((CH, 2, H), jnp.float32) for _ in range(DEPTH)],
            pltpu.VMEM_SHARED((N_ACC, H), jnp.float32),
            [pltpu.SemaphoreType.DMA for _ in range(DEPTH)],
        ],
    )
    def agg_kernel(xp_hbm, srcq_hbm, dsti_hbm, zeros_hbm, out_hbm,
                   srcq_iv, dsti_iv, bufs, acc_sh, gsems):
        c = lax.axis_index("c")
        s = lax.axis_index("s")
        wid = c * NS + s

        pltpu.sync_copy(zeros_hbm, acc_sh.at[pl.ds(s * ZROWS, ZROWS)])
        plsc.subcore_barrier()

        def seg_body(g, carry):
            pltpu.sync_copy(srcq_hbm.at[wid, pl.ds(g * SEG, SEG)], srcq_iv)
            pltpu.sync_copy(dsti_hbm.at[wid, pl.ds(g * SEG, SEG)], dsti_iv)
            for b in range(DEPTH):
                pltpu.async_copy(xp_hbm.at[srcq_iv.at[b]], bufs[b], gsems[b])

            def body(r, carry2):
                for b in range(DEPTH):
                    k = r * DEPTH + b
                    pltpu.make_async_copy(xp_hbm.at[srcq_iv.at[k]], bufs[b],
                                          gsems[b]).wait()
                    pltpu.sync_copy(bufs[b].reshape(2 * CH, H),
                                    acc_sh.at[dsti_iv.at[k]], add=True)
                    @pl.when(k + DEPTH < SEG)
                    def _():
                        pltpu.async_copy(xp_hbm.at[srcq_iv.at[k + DEPTH]],
                                         bufs[b], gsems[b])
                return carry2

            lax.fori_loop(0, SEG // DEPTH, body, 0)
            return carry

        lax.fori_loop(0, NSEG, seg_body, 0)
        plsc.subcore_barrier()

        pltpu.sync_copy(acc_sh.at[pl.ds(s * ZROWS, ZROWS)],
                        out_hbm.at[c, pl.ds(s * ZROWS, ZROWS)])

    return agg_kernel(xp, srcq, dsti, zeros)


def _tc_body(x_ref, agg_ref, w1_ref, b1_ref, w2_ref, b2_ref, g_ref, bt_ref,
             out_ref):
    h = x_ref[...] + agg_ref[0, :N] + agg_ref[1, :N]
    h = jnp.dot(h, w1_ref[...], preferred_element_type=jnp.float32)
    h = jnp.maximum(h + b1_ref[...], 0.0)
    h = jnp.dot(h, w2_ref[...], preferred_element_type=jnp.float32)
    h = h + b2_ref[...]
    mean = jnp.mean(h, axis=0, keepdims=True)
    var = jnp.mean((h - mean) * (h - mean), axis=0, keepdims=True)
    out_ref[...] = (h - mean) * lax.rsqrt(var + 1e-5) * g_ref[...] + bt_ref[...]


def kernel(x, edge_index, W1, b1, W2, b2, gamma, beta):
    src = edge_index[0].astype(jnp.int32)
    dst = edge_index[1].astype(jnp.int32)
    pad = E_PAD - E
    tile_of = (jnp.arange(E_PAD, dtype=jnp.int32) // (NCHUNK * CH)) % NS
    dump = N + tile_of
    src_p = jnp.concatenate([src, jnp.zeros((pad,), jnp.int32)])
    dst_p = jnp.concatenate([dst, jnp.zeros((pad,), jnp.int32)])
    valid = jnp.arange(E_PAD, dtype=jnp.int32) < E
    par = src_p % 2
    srcq = src_p // 2
    dste = jnp.where(valid & (par == 0), dst_p, dump)
    dsto = jnp.where(valid & (par == 1), dst_p, dump)
    dsti = jnp.stack([dste, dsto], axis=1).reshape(-1)
    srcq = srcq.reshape(NW, NCHUNK, CH)
    dsti = dsti.reshape(NW, NCHUNK, 2 * CH)
    zeros = jnp.zeros((ZROWS, H), jnp.float32)
    xp = x.reshape(N // 2, 2, H)

    agg = _sc_aggregate(xp, srcq, dsti, zeros)

    out = pl.pallas_call(
        _tc_body,
        out_shape=jax.ShapeDtypeStruct((N, H), jnp.float32),
    )(x, agg, W1, b1.reshape(1, H), W2, b2.reshape(1, H),
      gamma.reshape(1, H), beta.reshape(1, H))
    return out


# speedup vs baseline: 1.2469x; 1.2469x over previous
import functools

import jax
import jax.numpy as jnp
from jax import lax
from jax.experimental import pallas as pl
from jax.experimental.pallas import tpu as pltpu
from jax.experimental.pallas import tpu_sc as plsc

N = 10000
E = 320000
H = 128

NC = 2
NS = 16
NW = NC * NS
CH = 64
NCHUNK = 160
E_PAD = NW * NCHUNK * CH
DEPTH = 2
SEG = 16
NSEG = NCHUNK // SEG

N_ACC = 10112
ZROWS = N_ACC // NS


def _sc_aggregate(zt, zq, dstq, zeros):
    mesh = plsc.VectorSubcoreMesh(core_axis_name="c", subcore_axis_name="s")

    @functools.partial(
        pl.kernel,
        out_type=jax.ShapeDtypeStruct((NC, N_ACC // 2, 2, H), jnp.float32),
        mesh=mesh,
        scratch_types=[
            pltpu.VMEM((SEG, CH), jnp.int32),
            pltpu.VMEM((SEG, CH), jnp.int32),
            [pltpu.VMEM((CH, 2, H), jnp.float32) for _ in range(DEPTH)],
            pltpu.VMEM_SHARED((N_ACC // 2, 2, H), jnp.float32),
            [pltpu.SemaphoreType.DMA for _ in range(DEPTH)],
        ],
    )
    def agg_kernel(zt_hbm, zq_hbm, dstq_hbm, zeros_hbm, out_hbm,
                   zq_iv, dstq_iv, bufs, acc_sh, gsems):
        c = lax.axis_index("c")
        s = lax.axis_index("s")
        wid = c * NS + s

        pltpu.sync_copy(zeros_hbm, acc_sh.at[pl.ds(s * (ZROWS // 2), ZROWS // 2)])
        plsc.subcore_barrier()

        def seg_body(g, carry):
            pltpu.sync_copy(zq_hbm.at[wid, pl.ds(g * SEG, SEG)], zq_iv)
            pltpu.sync_copy(dstq_hbm.at[wid, pl.ds(g * SEG, SEG)], dstq_iv)
            for b in range(DEPTH):
                pltpu.async_copy(zt_hbm.at[zq_iv.at[b]], bufs[b], gsems[b])

            def body(r, carry2):
                for b in range(DEPTH):
                    k = r * DEPTH + b
                    pltpu.make_async_copy(zt_hbm.at[zq_iv.at[k]], bufs[b],
                                          gsems[b]).wait()
                    pltpu.sync_copy(bufs[b], acc_sh.at[dstq_iv.at[k]],
                                    add=True)
                    @pl.when(k + DEPTH < SEG)
                    def _():
                        pltpu.async_copy(zt_hbm.at[zq_iv.at[k + DEPTH]],
                                         bufs[b], gsems[b])
                return carry2

            lax.fori_loop(0, SEG // DEPTH, body, 0)
            return carry

        lax.fori_loop(0, NSEG, seg_body, 0)
        plsc.subcore_barrier()

        pltpu.sync_copy(acc_sh.at[pl.ds(s * (ZROWS // 2), ZROWS // 2)],
                        out_hbm.at[c, pl.ds(s * (ZROWS // 2), ZROWS // 2)])

    return agg_kernel(zt, zq, dstq, zeros)


def _tc_body(x_ref, agg_ref, w1_ref, b1_ref, w2_ref, b2_ref, g_ref, bt_ref,
             out_ref):
    h = x_ref[...] + agg_ref[0, :N] + agg_ref[1, :N]
    h = jnp.dot(h, w1_ref[...], preferred_element_type=jnp.float32)
    h = jnp.maximum(h + b1_ref[...], 0.0)
    h = jnp.dot(h, w2_ref[...], preferred_element_type=jnp.float32)
    h = h + b2_ref[...]
    mean = jnp.mean(h, axis=0, keepdims=True)
    var = jnp.mean((h - mean) * (h - mean), axis=0, keepdims=True)
    out_ref[...] = (h - mean) * lax.rsqrt(var + 1e-5) * g_ref[...] + bt_ref[...]


def kernel(x, edge_index, W1, b1, W2, b2, gamma, beta):
    src = edge_index[0].astype(jnp.int32)
    dst = edge_index[1].astype(jnp.int32)
    pad = E_PAD - E
    tile_of = (jnp.arange(E_PAD, dtype=jnp.int32) // (NCHUNK * CH)) % NS
    dump = N // 2 + tile_of
    src_p = jnp.concatenate([src, jnp.zeros((pad,), jnp.int32)])
    dst_p = jnp.concatenate([dst, jnp.zeros((pad,), jnp.int32)])
    valid = jnp.arange(E_PAD, dtype=jnp.int32) < E
    zq = 2 * src_p + (dst_p % 2)
    dstq = jnp.where(valid, dst_p // 2, dump)
    zq = zq.reshape(NW, NCHUNK, CH)
    dstq = dstq.reshape(NW, NCHUNK, CH)
    zeros = jnp.zeros((ZROWS // 2, 2, H), jnp.float32)
    zcol = jnp.zeros((N, 1, H), jnp.float32)
    xcol = x.reshape(N, 1, H)
    zt = jnp.concatenate([
        jnp.concatenate([xcol, zcol], axis=1),
        jnp.concatenate([zcol, xcol], axis=1),
    ], axis=1).reshape(2 * N, 2, H)

    agg = _sc_aggregate(zt, zq, dstq, zeros)
    agg = agg.reshape(NC, N_ACC, H)

    out = pl.pallas_call(
        _tc_body,
        out_shape=jax.ShapeDtypeStruct((N, H), jnp.float32),
    )(x, agg, W1, b1.reshape(1, H), W2, b2.reshape(1, H),
      gamma.reshape(1, H), beta.reshape(1, H))
    return out
